# baseline (speedup 1.0000x reference)
import functools

import jax
import jax.numpy as jnp
from jax import lax
from jax.experimental import pallas as pl
from jax.experimental.pallas import tpu as pltpu, tpu_sc as plsc

N = 10000
D = 128
E = 320000
M = 10240
NC = 2
NS = 16
NW = NC * NS
EW = E // NW
CW = 128
CH = 80
HCH = CH // 2
EWP = CH * CW
DUMP = N
RPT = M // NS
f32 = jnp.float32

_mesh = plsc.VectorSubcoreMesh(core_axis_name="c", subcore_axis_name="s")
_sc_params = pltpu.CompilerParams(needs_layout_passes=False,
                                  use_tc_tiling_on_sc=False)


@functools.partial(
    pl.kernel,
    out_type=jax.ShapeDtypeStruct((NW, M), f32),
    mesh=_mesh,
    scratch_types=[
        pltpu.VMEM((CH, CW), jnp.int32),
        pltpu.VMEM((M,), f32),
    ],
    compiler_params=_sc_params,
)
def _sc_deg(dst_hbm, out_hbm, idx_v, acc_v):
    c = lax.axis_index("c")
    s = lax.axis_index("s")
    wid = c * NS + s
    pltpu.sync_copy(dst_hbm.at[wid], idx_v)
    zeros = jnp.zeros((16,), f32)

    @pl.loop(0, M // 16)
    def _zero(i):
        acc_v[pl.ds(i * 16, 16)] = zeros

    ones = jnp.ones((16,), f32)

    @pl.loop(0, CH)
    def _chunk(j):
        @pl.loop(0, CW // 16)
        def _sub(q):
            di = idx_v[j, pl.ds(q * 16, 16)]
            plsc.addupdate_scatter(acc_v, [di], ones)

    pltpu.sync_copy(acc_v, out_hbm.at[wid])


def _tc_matmul(x_pad, W1):
    def body(x_ref, w_ref, o_ref):
        o_ref[...] = jnp.dot(x_ref[...], w_ref[...],
                             preferred_element_type=f32)

    return pl.pallas_call(
        body,
        grid=(8,),
        in_specs=[
            pl.BlockSpec((M // 8, D), lambda i: (i, 0)),
            pl.BlockSpec((D, D), lambda i: (0, 0)),
        ],
        out_specs=pl.BlockSpec((M // 8, D), lambda i: (i, 0)),
        out_shape=jax.ShapeDtypeStruct((M, D), f32),
    )(x_pad, W1)


def _tc_scale(degpT, y):
    def body(dp_ref, y_ref, dinv_ref, yp_ref):
        deg = jnp.sum(dp_ref[...], axis=1, keepdims=True) + 1.0
        dinv = lax.rsqrt(deg)
        dinv_ref[...] = dinv
        yp_ref[...] = y_ref[...] * dinv

    return pl.pallas_call(
        body,
        grid=(8,),
        in_specs=[
            pl.BlockSpec((M // 8, NW), lambda i: (i, 0)),
            pl.BlockSpec((M // 8, D), lambda i: (i, 0)),
        ],
        out_specs=[
            pl.BlockSpec((M // 8, 1), lambda i: (i, 0)),
            pl.BlockSpec((M // 8, D), lambda i: (i, 0)),
        ],
        out_shape=[
            jax.ShapeDtypeStruct((M, 1), f32),
            jax.ShapeDtypeStruct((M, D), f32),
        ],
    )(degpT, y)


@functools.partial(
    pl.kernel,
    out_type=jax.ShapeDtypeStruct((NC, M, D // 2), f32),
    mesh=_mesh,
    scratch_types=[
        pltpu.VMEM((HCH, CW), jnp.int32),
        pltpu.VMEM((HCH, CW), jnp.int32),
        pltpu.VMEM((CW, D // 2), f32),
        pltpu.VMEM((CW, D // 2), f32),
        pltpu.VMEM_SHARED((M, D // 2), f32),
        pltpu.VMEM_SHARED((M, D // 2), f32),
        pltpu.SemaphoreType.DMA,
        pltpu.SemaphoreType.DMA,
        pltpu.SemaphoreType.DMA,
        pltpu.SemaphoreType.DMA,
    ],
    compiler_params=_sc_params,
)
def _sc_agg(ypn_hbm, src_hbm, dst_hbm, out_hbm,
            idx_s, idx_d, rows0, rows1, yp_sh, acc_sh,
            gsem0, gsem1, ssem0, ssem1):
    c = lax.axis_index("c")
    s = lax.axis_index("s")

    zeros = jnp.zeros((16,), f32)

    @pl.loop(0, CW)
    def _zr(r):
        @pl.loop(0, D // 32)
        def _zq(q):
            rows0[r, pl.ds(q * 16, 16)] = zeros

    @pl.loop(0, RPT // CW)
    def _za(k):
        pltpu.sync_copy(rows0, acc_sh.at[pl.ds(s * RPT + k * CW, CW)])

    @pl.loop(0, RPT // CW)
    def _stage(k):
        off = s * RPT + k * CW
        pltpu.sync_copy(ypn_hbm.at[c].at[pl.ds(off, CW)], rows1)
        pltpu.sync_copy(rows1, yp_sh.at[pl.ds(off, CW)])

    plsc.subcore_barrier()

    for h in range(2):
        w = h * NS + s
        pltpu.sync_copy(src_hbm.at[w].at[pl.ds(0, HCH)], idx_s)
        pltpu.sync_copy(dst_hbm.at[w].at[pl.ds(0, HCH)], idx_d)
        pltpu.async_copy(yp_sh.at[idx_s.at[0]], rows0, gsem0)
        pltpu.async_copy(yp_sh.at[idx_s.at[1]], rows1, gsem1)

        @pl.loop(0, HCH, step=2)
        def _main(j):
            pltpu.make_async_copy(yp_sh.at[idx_s.at[j]], rows0, gsem0).wait()
            pltpu.async_copy(rows0, acc_sh.at[idx_d.at[j]], ssem0, add=True)
            pltpu.make_async_copy(
                yp_sh.at[idx_s.at[j + 1]], rows1, gsem1).wait()
            pltpu.async_copy(rows1, acc_sh.at[idx_d.at[j + 1]], ssem1,
                             add=True)
            pltpu.make_async_copy(rows0, acc_sh.at[idx_d.at[j]], ssem0).wait()

            @pl.when(j + 2 < HCH)
            def _g0():
                pltpu.async_copy(yp_sh.at[idx_s.at[j + 2]], rows0, gsem0)

            pltpu.make_async_copy(
                rows1, acc_sh.at[idx_d.at[j + 1]], ssem1).wait()

            @pl.when(j + 3 < HCH)
            def _g1():
                pltpu.async_copy(yp_sh.at[idx_s.at[j + 3]], rows1, gsem1)

        pltpu.sync_copy(src_hbm.at[w].at[pl.ds(HCH, HCH)], idx_s)
        pltpu.sync_copy(dst_hbm.at[w].at[pl.ds(HCH, HCH)], idx_d)
        pltpu.async_copy(yp_sh.at[idx_s.at[0]], rows0, gsem0)
        pltpu.async_copy(yp_sh.at[idx_s.at[1]], rows1, gsem1)

        @pl.loop(0, HCH, step=2)
        def _main2(j):
            pltpu.make_async_copy(yp_sh.at[idx_s.at[j]], rows0, gsem0).wait()
            pltpu.async_copy(rows0, acc_sh.at[idx_d.at[j]], ssem0, add=True)
            pltpu.make_async_copy(
                yp_sh.at[idx_s.at[j + 1]], rows1, gsem1).wait()
            pltpu.async_copy(rows1, acc_sh.at[idx_d.at[j + 1]], ssem1,
                             add=True)
            pltpu.make_async_copy(rows0, acc_sh.at[idx_d.at[j]], ssem0).wait()

            @pl.when(j + 2 < HCH)
            def _g0():
                pltpu.async_copy(yp_sh.at[idx_s.at[j + 2]], rows0, gsem0)

            pltpu.make_async_copy(
                rows1, acc_sh.at[idx_d.at[j + 1]], ssem1).wait()

            @pl.when(j + 3 < HCH)
            def _g1():
                pltpu.async_copy(yp_sh.at[idx_s.at[j + 3]], rows1, gsem1)

    plsc.subcore_barrier()

    @pl.loop(0, RPT // CW)
    def _wb(k):
        off = s * RPT + k * CW
        pltpu.sync_copy(acc_sh.at[pl.ds(off, CW)], rows0)
        pltpu.sync_copy(rows0, out_hbm.at[c].at[pl.ds(off, CW)])


def _tc_layer2(aggc, yp, dinv2d, b1row, w2row):
    def body(a_ref, yp_ref, dinv_ref, b1_ref, w2_ref, zp_ref):
        acc = a_ref[...] + yp_ref[...]
        out1 = dinv_ref[...] * acc + b1_ref[...]
        h = jnp.maximum(out1, 0.0)
        z = jnp.sum(h * w2_ref[...], axis=1, keepdims=True)
        zp_ref[...] = dinv_ref[...] * z

    return pl.pallas_call(
        body,
        grid=(8,),
        in_specs=[
            pl.BlockSpec((M // 8, D), lambda i: (i, 0)),
            pl.BlockSpec((M // 8, D), lambda i: (i, 0)),
            pl.BlockSpec((M // 8, 1), lambda i: (i, 0)),
            pl.BlockSpec((1, D), lambda i: (0, 0)),
            pl.BlockSpec((1, D), lambda i: (0, 0)),
        ],
        out_specs=pl.BlockSpec((M // 8, 1), lambda i: (i, 0)),
        out_shape=jax.ShapeDtypeStruct((M, 1), f32),
    )(aggc, yp, dinv2d, b1row, w2row)


@functools.partial(
    pl.kernel,
    out_type=jax.ShapeDtypeStruct((E,), f32),
    mesh=_mesh,
    scratch_types=[
        pltpu.VMEM((2, CH, CW), jnp.int32),
        pltpu.VMEM((2, CH, CW), jnp.int32),
        pltpu.VMEM((M,), f32),
        pltpu.VMEM((M,), f32),
        pltpu.VMEM((RPT,), f32),
        pltpu.VMEM((RPT,), f32),
        pltpu.VMEM((16,), f32),
        pltpu.VMEM((EWP,), f32),
        pltpu.VMEM((NS, RPT), f32),
        pltpu.VMEM_SHARED((NS, M), f32),
    ],
    compiler_params=_sc_params,
)
def _sc_scalar(zp_hbm, dinv_hbm, b2_hbm, src_hbm, dst_hbm, out_hbm,
               idx_s, idx_d, zp_v, acc_v, sl_v, dinv_v, b2_v, pr_v,
               buf16, part_sh):
    c = lax.axis_index("c")
    s = lax.axis_index("s")

    @pl.when(c == 0)
    def _active():
        pltpu.sync_copy(zp_hbm, zp_v)
        pltpu.sync_copy(dinv_hbm.at[pl.ds(s * RPT, RPT)], dinv_v)
        pltpu.sync_copy(b2_hbm, b2_v)
        pltpu.sync_copy(src_hbm.at[pl.ds(2 * s, 2)], idx_s)
        pltpu.sync_copy(dst_hbm.at[pl.ds(2 * s, 2)], idx_d)

        zeros = jnp.zeros((16,), f32)

        @pl.loop(0, M // 16)
        def _zero(i):
            acc_v[pl.ds(i * 16, 16)] = zeros

        for w in range(2):
            @pl.loop(0, CH)
            def _chunk(j):
                @pl.loop(0, CW // 16)
                def _sub(q):
                    si = idx_s[w, j, pl.ds(q * 16, 16)]
                    di = idx_d[w, j, pl.ds(q * 16, 16)]
                    vals = plsc.load_gather(zp_v, [si])
                    plsc.addupdate_scatter(acc_v, [di], vals)

        pltpu.sync_copy(acc_v, part_sh.at[s])
        plsc.subcore_barrier()
        pltpu.sync_copy(part_sh.at[:, pl.ds(s * RPT, RPT)], buf16)
        b2 = b2_v[...]

        @pl.loop(0, RPT // 16)
        def _o2(i):
            a = buf16[0, pl.ds(i * 16, 16)]
            for tt in range(1, NS):
                a = a + buf16[tt, pl.ds(i * 16, 16)]
            zv = zp_v[pl.ds(s * RPT + i * 16, 16)]
            dv = dinv_v[pl.ds(i * 16, 16)]
            sl_v[pl.ds(i * 16, 16)] = dv * (a + zv) + b2

        plsc.subcore_barrier()
        pltpu.sync_copy(sl_v, part_sh.at[0].at[pl.ds(s * RPT, RPT)])
        plsc.subcore_barrier()
        pltpu.sync_copy(part_sh.at[0], zp_v)

        for w in range(2):
            @pl.loop(0, CH)
            def _score(j):
                @pl.loop(0, CW // 16)
                def _sub(q):
                    si = idx_s[w, j, pl.ds(q * 16, 16)]
                    di = idx_d[w, j, pl.ds(q * 16, 16)]
                    a = plsc.load_gather(zp_v, [si])
                    b = plsc.load_gather(zp_v, [di])
                    xv = a * b
                    pr_v[pl.ds((j * (CW // 16) + q) * 16, 16)] = 1.0 / (1.0 + jnp.exp(-xv))

            pltpu.sync_copy(pr_v.at[pl.ds(0, EW)],
                            out_hbm.at[pl.ds((2 * s + w) * EW, EW)])


def kernel(x, edge_index, W1, b1, W2, b2):
    src = edge_index[0].reshape(NW, EW)
    dst = edge_index[1].reshape(NW, EW)
    pad_s = jnp.zeros((NW, EWP - EW), jnp.int32)
    pad_d = jnp.full((NW, EWP - EW), DUMP, jnp.int32)
    src_p = jnp.concatenate([src, pad_s], axis=1).reshape(NW, CH, CW)
    dst_p = jnp.concatenate([dst, pad_d], axis=1).reshape(NW, CH, CW)
    x_pad = jnp.concatenate([x, jnp.zeros((M - N, D), f32)], axis=0)
    b1row = b1.reshape(1, D)
    w2row = W2.reshape(1, D)
    b2v = jnp.broadcast_to(b2.reshape(1), (16,)).astype(f32)

    deg_p = _sc_deg(dst_p)
    y = _tc_matmul(x_pad, W1)
    dinv2d, yp = _tc_scale(deg_p.T, y)
    ypn = jnp.stack([yp[:, :D // 2], yp[:, D // 2:]])
    parts = _sc_agg(ypn, src_p, dst_p)
    aggc = jnp.concatenate([parts[0], parts[1]], axis=1)
    zp2d = _tc_layer2(aggc, yp, dinv2d, b1row, w2row)
    probs = _sc_scalar(zp2d.reshape(M), dinv2d.reshape(M), b2v, src_p, dst_p)
    return probs


# speedup vs baseline: 1.0492x; 1.0492x over previous
import functools

import jax
import jax.numpy as jnp
from jax import lax
from jax.experimental import pallas as pl
from jax.experimental.pallas import tpu as pltpu, tpu_sc as plsc

N = 10000
D = 128
E = 320000
M = 10240
NC = 2
NS = 16
NW = NC * NS
EW = E // NW
CW = 128
CH = 80
HCH = CH // 2
EWP = CH * CW
DUMP = N
RPT = M // NS
f32 = jnp.float32

_mesh = plsc.VectorSubcoreMesh(core_axis_name="c", subcore_axis_name="s")
_sc_params = pltpu.CompilerParams(needs_layout_passes=False,
                                  use_tc_tiling_on_sc=False)


@functools.partial(
    pl.kernel,
    out_type=jax.ShapeDtypeStruct((NW, M), f32),
    mesh=_mesh,
    scratch_types=[
        pltpu.VMEM((CH, CW), jnp.int32),
        pltpu.VMEM((M,), f32),
    ],
    compiler_params=_sc_params,
)
def _sc_deg(dst_hbm, out_hbm, idx_v, acc_v):
    c = lax.axis_index("c")
    s = lax.axis_index("s")
    wid = c * NS + s
    pltpu.sync_copy(dst_hbm.at[wid], idx_v)
    zeros = jnp.zeros((16,), f32)

    @pl.loop(0, M // 16)
    def _zero(i):
        acc_v[pl.ds(i * 16, 16)] = zeros

    ones = jnp.ones((16,), f32)

    @pl.loop(0, CH)
    def _chunk(j):
        @pl.loop(0, CW // 16)
        def _sub(q):
            di = idx_v[j, pl.ds(q * 16, 16)]
            plsc.addupdate_scatter(acc_v, [di], ones)

    pltpu.sync_copy(acc_v, out_hbm.at[wid])


def _tc_mm_scale(x_pad, W1, degpT):
    def body(x_ref, w_ref, dp_ref, dinv_ref, ypn_ref):
        y = jnp.dot(x_ref[...], w_ref[...], preferred_element_type=f32)
        deg = jnp.sum(dp_ref[...], axis=1, keepdims=True) + 1.0
        dinv = lax.rsqrt(deg)
        dinv_ref[...] = dinv
        yp = y * dinv
        ypn_ref[0] = yp[:, : D // 2]
        ypn_ref[1] = yp[:, D // 2 :]

    return pl.pallas_call(
        body,
        grid=(8,),
        in_specs=[
            pl.BlockSpec((M // 8, D), lambda i: (i, 0)),
            pl.BlockSpec((D, D), lambda i: (0, 0)),
            pl.BlockSpec((M // 8, NW), lambda i: (i, 0)),
        ],
        out_specs=[
            pl.BlockSpec((M // 8, 1), lambda i: (i, 0)),
            pl.BlockSpec((NC, M // 8, D // 2), lambda i: (0, i, 0)),
        ],
        out_shape=[
            jax.ShapeDtypeStruct((M, 1), f32),
            jax.ShapeDtypeStruct((NC, M, D // 2), f32),
        ],
    )(x_pad, W1, degpT)


@functools.partial(
    pl.kernel,
    out_type=jax.ShapeDtypeStruct((NC, M, D // 2), f32),
    mesh=_mesh,
    scratch_types=[
        pltpu.VMEM((HCH, CW), jnp.int32),
        pltpu.VMEM((HCH, CW), jnp.int32),
        pltpu.VMEM((CW, D // 2), f32),
        pltpu.VMEM((CW, D // 2), f32),
        pltpu.VMEM_SHARED((M, D // 2), f32),
        pltpu.VMEM_SHARED((M, D // 2), f32),
        pltpu.SemaphoreType.DMA,
        pltpu.SemaphoreType.DMA,
        pltpu.SemaphoreType.DMA,
        pltpu.SemaphoreType.DMA,
    ],
    compiler_params=_sc_params,
)
def _sc_agg(ypn_hbm, src_hbm, dst_hbm, out_hbm,
            idx_s, idx_d, rows0, rows1, yp_sh, acc_sh,
            gsem0, gsem1, ssem0, ssem1):
    c = lax.axis_index("c")
    s = lax.axis_index("s")

    zeros = jnp.zeros((16,), f32)

    @pl.loop(0, CW)
    def _zr(r):
        @pl.loop(0, D // 32)
        def _zq(q):
            rows0[r, pl.ds(q * 16, 16)] = zeros

    @pl.loop(0, RPT // CW)
    def _za(k):
        pltpu.sync_copy(rows0, acc_sh.at[pl.ds(s * RPT + k * CW, CW)])

    @pl.loop(0, RPT // CW)
    def _stage(k):
        off = s * RPT + k * CW
        pltpu.sync_copy(ypn_hbm.at[c].at[pl.ds(off, CW)], rows1)
        pltpu.sync_copy(rows1, yp_sh.at[pl.ds(off, CW)])

    plsc.subcore_barrier()

    for h in range(2):
        w = h * NS + s
        pltpu.sync_copy(src_hbm.at[w].at[pl.ds(0, HCH)], idx_s)
        pltpu.sync_copy(dst_hbm.at[w].at[pl.ds(0, HCH)], idx_d)
        pltpu.async_copy(yp_sh.at[idx_s.at[0]], rows0, gsem0)
        pltpu.async_copy(yp_sh.at[idx_s.at[1]], rows1, gsem1)

        @pl.loop(0, HCH, step=2)
        def _main(j):
            pltpu.make_async_copy(yp_sh.at[idx_s.at[j]], rows0, gsem0).wait()
            pltpu.async_copy(rows0, acc_sh.at[idx_d.at[j]], ssem0, add=True)
            pltpu.make_async_copy(
                yp_sh.at[idx_s.at[j + 1]], rows1, gsem1).wait()
            pltpu.async_copy(rows1, acc_sh.at[idx_d.at[j + 1]], ssem1,
                             add=True)
            pltpu.make_async_copy(rows0, acc_sh.at[idx_d.at[j]], ssem0).wait()

            @pl.when(j + 2 < HCH)
            def _g0():
                pltpu.async_copy(yp_sh.at[idx_s.at[j + 2]], rows0, gsem0)

            pltpu.make_async_copy(
                rows1, acc_sh.at[idx_d.at[j + 1]], ssem1).wait()

            @pl.when(j + 3 < HCH)
            def _g1():
                pltpu.async_copy(yp_sh.at[idx_s.at[j + 3]], rows1, gsem1)

        pltpu.sync_copy(src_hbm.at[w].at[pl.ds(HCH, HCH)], idx_s)
        pltpu.sync_copy(dst_hbm.at[w].at[pl.ds(HCH, HCH)], idx_d)
        pltpu.async_copy(yp_sh.at[idx_s.at[0]], rows0, gsem0)
        pltpu.async_copy(yp_sh.at[idx_s.at[1]], rows1, gsem1)

        @pl.loop(0, HCH, step=2)
        def _main2(j):
            pltpu.make_async_copy(yp_sh.at[idx_s.at[j]], rows0, gsem0).wait()
            pltpu.async_copy(rows0, acc_sh.at[idx_d.at[j]], ssem0, add=True)
            pltpu.make_async_copy(
                yp_sh.at[idx_s.at[j + 1]], rows1, gsem1).wait()
            pltpu.async_copy(rows1, acc_sh.at[idx_d.at[j + 1]], ssem1,
                             add=True)
            pltpu.make_async_copy(rows0, acc_sh.at[idx_d.at[j]], ssem0).wait()

            @pl.when(j + 2 < HCH)
            def _g0():
                pltpu.async_copy(yp_sh.at[idx_s.at[j + 2]], rows0, gsem0)

            pltpu.make_async_copy(
                rows1, acc_sh.at[idx_d.at[j + 1]], ssem1).wait()

            @pl.when(j + 3 < HCH)
            def _g1():
                pltpu.async_copy(yp_sh.at[idx_s.at[j + 3]], rows1, gsem1)

    plsc.subcore_barrier()

    @pl.loop(0, RPT // CW)
    def _wb(k):
        off = s * RPT + k * CW
        pltpu.sync_copy(acc_sh.at[pl.ds(off, CW)], rows0)
        pltpu.sync_copy(rows0, out_hbm.at[c].at[pl.ds(off, CW)])


def _tc_layer2(parts, ypn, dinv2d, b1n, w2n):
    def body(p_ref, yp_ref, dinv_ref, b1_ref, w2_ref, zp_ref):
        dinv = dinv_ref[...]
        b1v = b1_ref[...]
        w2v = w2_ref[...]
        z = dinv * 0.0
        for h in range(2):
            out1 = dinv * (p_ref[h] + yp_ref[h]) + b1v[h][None, :]
            hh = jnp.maximum(out1, 0.0)
            z = z + jnp.sum(hh * w2v[h][None, :], axis=1, keepdims=True)
        zp_ref[...] = dinv * z

    return pl.pallas_call(
        body,
        grid=(8,),
        in_specs=[
            pl.BlockSpec((NC, M // 8, D // 2), lambda i: (0, i, 0)),
            pl.BlockSpec((NC, M // 8, D // 2), lambda i: (0, i, 0)),
            pl.BlockSpec((M // 8, 1), lambda i: (i, 0)),
            pl.BlockSpec((NC, D // 2), lambda i: (0, 0)),
            pl.BlockSpec((NC, D // 2), lambda i: (0, 0)),
        ],
        out_specs=pl.BlockSpec((M // 8, 1), lambda i: (i, 0)),
        out_shape=jax.ShapeDtypeStruct((M, 1), f32),
    )(parts, ypn, dinv2d, b1n, w2n)


@functools.partial(
    pl.kernel,
    out_type=jax.ShapeDtypeStruct((E,), f32),
    mesh=_mesh,
    scratch_types=[
        pltpu.VMEM((2, CH, CW), jnp.int32),
        pltpu.VMEM((2, CH, CW), jnp.int32),
        pltpu.VMEM((M,), f32),
        pltpu.VMEM((M,), f32),
        pltpu.VMEM((RPT,), f32),
        pltpu.VMEM((RPT,), f32),
        pltpu.VMEM((16,), f32),
        pltpu.VMEM((EWP,), f32),
        pltpu.VMEM((NS, RPT), f32),
        pltpu.VMEM_SHARED((NS, M), f32),
    ],
    compiler_params=_sc_params,
)
def _sc_scalar(zp_hbm, dinv_hbm, b2_hbm, src_hbm, dst_hbm, out_hbm,
               idx_s, idx_d, zp_v, acc_v, sl_v, dinv_v, b2_v, pr_v,
               buf16, part_sh):
    c = lax.axis_index("c")
    s = lax.axis_index("s")

    @pl.when(c == 0)
    def _active():
        pltpu.sync_copy(zp_hbm, zp_v)
        pltpu.sync_copy(dinv_hbm.at[pl.ds(s * RPT, RPT)], dinv_v)
        pltpu.sync_copy(b2_hbm, b2_v)
        pltpu.sync_copy(src_hbm.at[pl.ds(2 * s, 2)], idx_s)
        pltpu.sync_copy(dst_hbm.at[pl.ds(2 * s, 2)], idx_d)

        zeros = jnp.zeros((16,), f32)

        @pl.loop(0, M // 16)
        def _zero(i):
            acc_v[pl.ds(i * 16, 16)] = zeros

        for w in range(2):
            @pl.loop(0, CH)
            def _chunk(j):
                @pl.loop(0, CW // 16)
                def _sub(q):
                    si = idx_s[w, j, pl.ds(q * 16, 16)]
                    di = idx_d[w, j, pl.ds(q * 16, 16)]
                    vals = plsc.load_gather(zp_v, [si])
                    plsc.addupdate_scatter(acc_v, [di], vals)

        pltpu.sync_copy(acc_v, part_sh.at[s])
        plsc.subcore_barrier()
        pltpu.sync_copy(part_sh.at[:, pl.ds(s * RPT, RPT)], buf16)
        b2 = b2_v[...]

        @pl.loop(0, RPT // 16)
        def _o2(i):
            a = buf16[0, pl.ds(i * 16, 16)]
            for tt in range(1, NS):
                a = a + buf16[tt, pl.ds(i * 16, 16)]
            zv = zp_v[pl.ds(s * RPT + i * 16, 16)]
            dv = dinv_v[pl.ds(i * 16, 16)]
            sl_v[pl.ds(i * 16, 16)] = dv * (a + zv) + b2

        plsc.subcore_barrier()
        pltpu.sync_copy(sl_v, part_sh.at[0].at[pl.ds(s * RPT, RPT)])
        plsc.subcore_barrier()
        pltpu.sync_copy(part_sh.at[0], zp_v)

        for w in range(2):
            @pl.loop(0, CH)
            def _score(j):
                @pl.loop(0, CW // 16)
                def _sub(q):
                    si = idx_s[w, j, pl.ds(q * 16, 16)]
                    di = idx_d[w, j, pl.ds(q * 16, 16)]
                    a = plsc.load_gather(zp_v, [si])
                    b = plsc.load_gather(zp_v, [di])
                    xv = a * b
                    pr_v[pl.ds((j * (CW // 16) + q) * 16, 16)] = 1.0 / (1.0 + jnp.exp(-xv))

            pltpu.sync_copy(pr_v.at[pl.ds(0, EW)],
                            out_hbm.at[pl.ds((2 * s + w) * EW, EW)])


def kernel(x, edge_index, W1, b1, W2, b2):
    src = edge_index[0].reshape(NW, EW)
    dst = edge_index[1].reshape(NW, EW)
    pad_s = jnp.zeros((NW, EWP - EW), jnp.int32)
    pad_d = jnp.full((NW, EWP - EW), DUMP, jnp.int32)
    src_p = jnp.concatenate([src, pad_s], axis=1).reshape(NW, CH, CW)
    dst_p = jnp.concatenate([dst, pad_d], axis=1).reshape(NW, CH, CW)
    x_pad = jnp.concatenate([x, jnp.zeros((M - N, D), f32)], axis=0)
    b1n = b1.reshape(NC, D // 2)
    w2n = W2.reshape(NC, D // 2)
    b2v = jnp.broadcast_to(b2.reshape(1), (16,)).astype(f32)

    deg_p = _sc_deg(dst_p)
    dinv2d, ypn = _tc_mm_scale(x_pad, W1, deg_p.T)
    parts = _sc_agg(ypn, src_p, dst_p)
    zp2d = _tc_layer2(parts, ypn, dinv2d, b1n, w2n)
    probs = _sc_scalar(zp2d.reshape(M), dinv2d.reshape(M), b2v, src_p, dst_p)
    return probs
